# baseline (speedup 1.0000x reference)
import functools

import jax
import jax.numpy as jnp
from jax import lax
from jax.experimental import pallas as pl
from jax.experimental.pallas import tpu as pltpu
from jax.experimental.pallas import tpu_sc as plsc

_NOISE_STD = 0.05
_B, _D = 16384, 64
_NC, _NS, _L = 2, 16, 16
_NW = _NC * _NS
_ROWS = _B // _NW
_T = 1001
_TR = _D // 8
_TC = _B // 128
_WTC = _TC // _NW
_SEG = _WTC * 1024
_TROW = 128 * 1024

_mesh = plsc.VectorSubcoreMesh(core_axis_name="c", subcore_axis_name="s")


@functools.partial(
    pl.kernel,
    mesh=_mesh,
    out_type=(
        jax.ShapeDtypeStruct((_B * _D,), jnp.float32),
        jax.ShapeDtypeStruct((_B,), jnp.int32),
    ),
    compiler_params=pltpu.CompilerParams(needs_layout_passes=False,
                                         use_tc_tiling_on_sc=False),
    scratch_types=[
        pltpu.VMEM((_ROWS,), jnp.int32),
        pltpu.VMEM((_T,), jnp.float32),
        pltpu.VMEM((_T,), jnp.float32),
        pltpu.VMEM((_ROWS,), jnp.float32),
        pltpu.VMEM((_ROWS,), jnp.float32),
        pltpu.VMEM((4, _SEG), jnp.float32),
        pltpu.VMEM((4, _SEG), jnp.float32),
        pltpu.VMEM((2, _SEG), jnp.float32),
        pltpu.SemaphoreType.DMA,
        pltpu.SemaphoreType.DMA,
        pltpu.SemaphoreType.DMA,
        pltpu.SemaphoreType.DMA,
        pltpu.SemaphoreType.DMA,
        pltpu.SemaphoreType.DMA,
        pltpu.SemaphoreType.DMA,
        pltpu.SemaphoreType.DMA,
        pltpu.SemaphoreType.DMA,
        pltpu.SemaphoreType.DMA,
        pltpu.SemaphoreType.DMA,
    ],
)
def _q_sample_sc(x0_hbm, a_hbm, b_hbm, t_hbm, nz_hbm, out_hbm, tout_hbm,
                 t_v, a_v, b_v, at_v, ct_v, x0_v, nz_v, o_v,
                 sx0, sx1, sx2, sx3, snz0, snz1, snz2, snz3, so0, so1, ssm):
    wid = lax.axis_index("s") * _NC + lax.axis_index("c")
    base = wid * _ROWS
    sx = (sx0, sx1, sx2, sx3)
    snz = (snz0, snz1, snz2, snz3)
    so = (so0, so1)

    def start_in(k, p):
        off = k * _TROW + wid * _SEG
        pltpu.make_async_copy(x0_hbm.at[pl.ds(off, _SEG)], x0_v.at[p],
                              sx[p]).start()
        pltpu.make_async_copy(nz_hbm.at[pl.ds(off, _SEG)], nz_v.at[p],
                              snz[p]).start()

    def wait_in(p):
        pltpu.make_async_copy(x0_hbm.at[pl.ds(0, _SEG)], x0_v.at[p],
                              sx[p]).wait()
        pltpu.make_async_copy(nz_hbm.at[pl.ds(0, _SEG)], nz_v.at[p],
                              snz[p]).wait()

    def start_out(k, p):
        off = k * _TROW + wid * _SEG
        pltpu.make_async_copy(o_v.at[p], out_hbm.at[pl.ds(off, _SEG)],
                              so[p]).start()

    def wait_out(p):
        pltpu.make_async_copy(o_v.at[p], out_hbm.at[pl.ds(0, _SEG)],
                              so[p]).wait()

    start_in(0, 0)
    start_in(1, 1)

    ct = pltpu.make_async_copy(t_hbm.at[pl.ds(base, _ROWS)], t_v, ssm)
    ca = pltpu.make_async_copy(a_hbm, a_v, ssm)
    cb = pltpu.make_async_copy(b_hbm, b_v, ssm)
    for c in (ct, ca, cb):
        c.start()
    for c in (ct, ca, cb):
        c.wait()

    std = jnp.float32(_NOISE_STD)

    def gather_body(g, _):
        sl = pl.ds(g * _L, _L)
        idx = t_v[sl]
        at_v[sl] = plsc.load_gather(a_v, [idx])
        ct_v[sl] = plsc.load_gather(b_v, [idx]) * std
        return _

    lax.fori_loop(0, _ROWS // _L, gather_body, None, unroll=2)

    ctout = pltpu.make_async_copy(t_v, tout_hbm.at[pl.ds(base, _ROWS)], ssm)
    ctout.start()

    def compute_block(p, blk, _):
        csl = pl.ds(blk * _L, _L)
        av = at_v[csl]
        cv = ct_v[csl]
        tile, lane0 = blk // 8, (blk % 8) * _L
        for sub in range(8):
            dsl = pl.ds(tile * 1024 + sub * 128 + lane0, _L)
            o_v[p, dsl] = av * x0_v[p, dsl] + cv * nz_v[p, dsl]
        return _

    def pair_body(i, _):
        for p in (0, 1):
            k = 2 * i + p
            wait_in(p)
            pl.when(i > 0)(lambda: wait_out(p))

            @plsc.parallel_loop(0, _SEG // 128, unroll=4)
            def _(blk, _p=p):
                compute_block(_p, blk, None)
            start_out(k, p)
            pl.when(i < _TR // 2 - 1)(lambda: start_in(k + 2, p))
        return _

    lax.fori_loop(0, _TR // 2, pair_body, None)
    ctout.wait()
    wait_out(0)
    wait_out(1)


def _phys_view(x):
    return x.T.reshape(_D // 8, 8, _B // 128, 128).transpose(0, 2, 1, 3).reshape(-1)


def kernel(x_0, alphas_bar_sqrt, one_minus_alphas_bar_sqrt, t, noise):
    t32 = t.astype(jnp.int32)
    x_t_flat, t_out = _q_sample_sc(
        _phys_view(x_0), alphas_bar_sqrt, one_minus_alphas_bar_sqrt,
        t32, _phys_view(noise))
    x_t = (x_t_flat.reshape(_D // 8, _B // 128, 8, 128)
           .transpose(0, 2, 1, 3).reshape(_D, _B).T)
    return (x_t, t_out.reshape(_B, 1))


# speedup vs baseline: 1.0035x; 1.0035x over previous
import functools

import jax
import jax.numpy as jnp
from jax import lax
from jax.experimental import pallas as pl
from jax.experimental.pallas import tpu as pltpu
from jax.experimental.pallas import tpu_sc as plsc

_NOISE_STD = 0.05
_B, _D = 16384, 64
_NC, _NS, _L = 2, 16, 16
_NW = _NC * _NS
_ROWS = _B // _NW
_T = 1001
_TR = _D // 8
_TC = _B // 128
_WTC = _TC // _NW
_SEG = _WTC * 1024
_TROW = 128 * 1024

_mesh = plsc.VectorSubcoreMesh(core_axis_name="c", subcore_axis_name="s")


@functools.partial(
    pl.kernel,
    mesh=_mesh,
    out_type=(
        jax.ShapeDtypeStruct((_B * _D,), jnp.float32),
        jax.ShapeDtypeStruct((_B,), jnp.int32),
    ),
    compiler_params=pltpu.CompilerParams(needs_layout_passes=False,
                                         use_tc_tiling_on_sc=False),
    scratch_types=[
        pltpu.VMEM((_ROWS,), jnp.int32),
        pltpu.VMEM((_T,), jnp.float32),
        pltpu.VMEM((_T,), jnp.float32),
        pltpu.VMEM((_ROWS,), jnp.float32),
        pltpu.VMEM((_ROWS,), jnp.float32),
        pltpu.VMEM((4, _SEG), jnp.float32),
        pltpu.VMEM((4, _SEG), jnp.float32),
        pltpu.VMEM((2, _SEG), jnp.float32),
        pltpu.SemaphoreType.DMA,
        pltpu.SemaphoreType.DMA,
        pltpu.SemaphoreType.DMA,
        pltpu.SemaphoreType.DMA,
        pltpu.SemaphoreType.DMA,
        pltpu.SemaphoreType.DMA,
        pltpu.SemaphoreType.DMA,
        pltpu.SemaphoreType.DMA,
        pltpu.SemaphoreType.DMA,
        pltpu.SemaphoreType.DMA,
        pltpu.SemaphoreType.DMA,
    ],
)
def _q_sample_sc(x0_hbm, a_hbm, b_hbm, t_hbm, nz_hbm, out_hbm, tout_hbm,
                 t_v, a_v, b_v, at_v, ct_v, x0_v, nz_v, o_v,
                 sx0, sx1, sx2, sx3, snz0, snz1, snz2, snz3, so0, so1, ssm):
    wid = lax.axis_index("s") * _NC + lax.axis_index("c")
    base = wid * _ROWS
    sx = (sx0, sx1, sx2, sx3)
    snz = (snz0, snz1, snz2, snz3)
    so = (so0, so1)

    def start_in(k, p):
        off = k * _TROW + wid * _SEG
        pltpu.make_async_copy(x0_hbm.at[pl.ds(off, _SEG)], x0_v.at[p],
                              sx[p]).start()
        pltpu.make_async_copy(nz_hbm.at[pl.ds(off, _SEG)], nz_v.at[p],
                              snz[p]).start()

    def wait_in(p):
        pltpu.make_async_copy(x0_hbm.at[pl.ds(0, _SEG)], x0_v.at[p],
                              sx[p]).wait()
        pltpu.make_async_copy(nz_hbm.at[pl.ds(0, _SEG)], nz_v.at[p],
                              snz[p]).wait()

    def start_out(k, p):
        off = k * _TROW + wid * _SEG
        pltpu.make_async_copy(o_v.at[p], out_hbm.at[pl.ds(off, _SEG)],
                              so[p]).start()

    def wait_out(p):
        pltpu.make_async_copy(o_v.at[p], out_hbm.at[pl.ds(0, _SEG)],
                              so[p]).wait()

    start_in(0, 0)
    start_in(1, 1)

    ct = pltpu.make_async_copy(t_hbm.at[pl.ds(base, _ROWS)], t_v, ssm)
    ca = pltpu.make_async_copy(a_hbm, a_v, ssm)
    cb = pltpu.make_async_copy(b_hbm, b_v, ssm)
    for c in (ct, ca, cb):
        c.start()
    for c in (ct, ca, cb):
        c.wait()

    std = jnp.float32(_NOISE_STD)

    def gather_body(g, _):
        sl = pl.ds(g * _L, _L)
        idx = t_v[sl]
        at_v[sl] = plsc.load_gather(a_v, [idx])
        ct_v[sl] = plsc.load_gather(b_v, [idx]) * std
        return _

    lax.fori_loop(0, _ROWS // _L, gather_body, None, unroll=2)

    ctout = pltpu.make_async_copy(t_v, tout_hbm.at[pl.ds(base, _ROWS)], ssm)
    ctout.start()

    def compute_block(p, blk, _):
        csl = pl.ds(blk * _L, _L)
        av = at_v[csl]
        cv = ct_v[csl]
        tile, lane0 = blk // 8, (blk % 8) * _L
        for sub in range(8):
            dsl = pl.ds(tile * 1024 + sub * 128 + lane0, _L)
            o_v[p, dsl] = av * x0_v[p, dsl] + cv * nz_v[p, dsl]
        return _

    def pair_body(i, _):
        for p in (0, 1):
            k = 2 * i + p
            wait_in(p)
            pl.when(i > 0)(lambda: wait_out(p))

            @plsc.parallel_loop(0, _SEG // 128, unroll=2)
            def _(blk, _p=p):
                compute_block(_p, blk, None)
            start_out(k, p)
            pl.when(i < _TR // 2 - 1)(lambda: start_in(k + 2, p))
        return _

    lax.fori_loop(0, _TR // 2, pair_body, None)
    ctout.wait()
    wait_out(0)
    wait_out(1)


def _phys_view(x):
    return x.T.reshape(_D // 8, 8, _B // 128, 128).transpose(0, 2, 1, 3).reshape(-1)


def kernel(x_0, alphas_bar_sqrt, one_minus_alphas_bar_sqrt, t, noise):
    t32 = t.astype(jnp.int32)
    x_t_flat, t_out = _q_sample_sc(
        _phys_view(x_0), alphas_bar_sqrt, one_minus_alphas_bar_sqrt,
        t32, _phys_view(noise))
    x_t = (x_t_flat.reshape(_D // 8, _B // 128, 8, 128)
           .transpose(0, 2, 1, 3).reshape(_D, _B).T)
    return (x_t, t_out.reshape(_B, 1))
